# Initial kernel scaffold; baseline (speedup 1.0000x reference)
import jax
import jax.numpy as jnp
from jax.experimental import pallas as pl


def kernel(x, edge_index, W, b):
    raise NotImplementedError("write your pallas kernel here")


# speedup vs baseline: 4.0825x; 4.0825x over previous
import functools

import jax
import jax.numpy as jnp
from jax import lax
from jax.experimental import pallas as pl
from jax.experimental.pallas import tpu as pltpu
from jax.experimental.pallas import tpu_sc as plsc

N = 10000
D = 128
DPAD = 144
N_ACC = 10112
CHUNK = 128
NC = 2
NS = 16
NW = NC * NS


def _make_sc_agg(K):
  mesh = plsc.VectorSubcoreMesh(core_axis_name="c", subcore_axis_name="s")

  @functools.partial(
      pl.kernel,
      mesh=mesh,
      out_type=jax.ShapeDtypeStruct((NC, N_ACC, DPAD), jnp.float32),
      scratch_types=[
          pltpu.VMEM_SHARED((N_ACC, DPAD), jnp.float32),
          pltpu.VMEM((K, CHUNK), jnp.int32),
          pltpu.VMEM((K, CHUNK), jnp.int32),
          pltpu.VMEM((CHUNK, DPAD), jnp.float32),
          pltpu.SemaphoreType.DMA,
      ],
      compiler_params=pltpu.CompilerParams(use_tc_tiling_on_sc=False),
  )
  def sc_agg(xaug_hbm, src_hbm, dst_hbm, zeros_hbm, out_hbm, acc_sh, src_v,
             dst_v, rows_v, sem):
    c = lax.axis_index("c")
    s = lax.axis_index("s")
    wid = c * NS + s

    zrows = N_ACC // NS
    pltpu.sync_copy(
        zeros_hbm.at[pl.ds(s * zrows, zrows)],
        acc_sh.at[pl.ds(s * zrows, zrows)],
    )

    pltpu.sync_copy(src_hbm.at[pl.ds(wid * K, K)], src_v)
    pltpu.sync_copy(dst_hbm.at[pl.ds(wid * K, K)], dst_v)
    plsc.subcore_barrier()

    @pl.loop(0, K)
    def _(k):
      pltpu.async_copy(xaug_hbm.at[src_v.at[k]], rows_v, sem).wait()
      pltpu.sync_copy(rows_v, acc_sh.at[dst_v.at[k]], add=True)

    plsc.subcore_barrier()

    orows = N_ACC // NS
    pltpu.sync_copy(
        acc_sh.at[pl.ds(s * orows, orows)],
        out_hbm.at[c, pl.ds(s * orows, orows)],
    )

  return sc_agg


BN = 2000


def _tc_body(p_ref, x_ref, w_ref, b_ref, o_ref):
  p = p_ref[0] + p_ref[1]
  deg = p[:, D:D + 1]
  agg = p[:, :D] / jnp.maximum(deg, 1.0)
  h = jnp.dot(agg, w_ref[...], preferred_element_type=jnp.float32) + b_ref[...]
  o_ref[...] = jnp.maximum(h, 0.0) + x_ref[...]


def _tc_update(partials, x, W, b2):
  return pl.pallas_call(
      _tc_body,
      grid=(N // BN,),
      in_specs=[
          pl.BlockSpec((NC, BN, DPAD), lambda i: (0, i, 0)),
          pl.BlockSpec((BN, D), lambda i: (i, 0)),
          pl.BlockSpec((D, D), lambda i: (0, 0)),
          pl.BlockSpec((1, D), lambda i: (0, 0)),
      ],
      out_specs=pl.BlockSpec((BN, D), lambda i: (i, 0)),
      out_shape=jax.ShapeDtypeStruct((N, D), jnp.float32),
  )(partials, x, W, b2)


@jax.jit
def kernel(x, edge_index, W, b):
  E = edge_index.shape[1]
  K = -(-E // (NW * CHUNK))
  K = -(-K // 8) * 8
  E_pad = NW * CHUNK * K
  pad = E_pad - E
  src = jnp.concatenate([edge_index[0], jnp.zeros((pad,), jnp.int32)])
  dst = jnp.concatenate([edge_index[1], jnp.full((pad,), N, jnp.int32)])
  src2 = src.reshape(NW * K, CHUNK)
  dst2 = dst.reshape(NW * K, CHUNK)
  xaug = jnp.concatenate(
      [x, jnp.ones((N, 1), jnp.float32), jnp.zeros((N, 15), jnp.float32)],
      axis=1)
  zeros = jnp.zeros((N_ACC, DPAD), jnp.float32)
  partials = _make_sc_agg(K)(xaug, src2, dst2, zeros)
  return _tc_update(partials, x, W, b.reshape(1, D))


# speedup vs baseline: 4.3548x; 1.0667x over previous
import functools

import jax
import jax.numpy as jnp
from jax import lax
from jax.experimental import pallas as pl
from jax.experimental.pallas import tpu as pltpu
from jax.experimental.pallas import tpu_sc as plsc

N = 10000
D = 128
DPAD = 144
N_ACC = 10112
CHUNK = 128
NC = 2
NS = 16
NW = NC * NS
NBUF = 2
IB = 8


def _make_sc_agg(K):
  mesh = plsc.VectorSubcoreMesh(core_axis_name="c", subcore_axis_name="s")

  @functools.partial(
      pl.kernel,
      mesh=mesh,
      out_type=jax.ShapeDtypeStruct((NC, N_ACC, DPAD), jnp.float32),
      scratch_types=[
          pltpu.VMEM_SHARED((N_ACC, DPAD), jnp.float32),
          pltpu.VMEM((IB, CHUNK), jnp.int32),
          pltpu.VMEM((IB, CHUNK), jnp.int32),
          pltpu.VMEM((NBUF, CHUNK, DPAD), jnp.float32),
      ] + [pltpu.SemaphoreType.DMA] * (2 * NBUF),
      compiler_params=pltpu.CompilerParams(use_tc_tiling_on_sc=False),
  )
  def sc_agg(xaug_hbm, src_hbm, dst_hbm, zeros_hbm, out_hbm, acc_sh, src_v,
             dst_v, rows_v, *sems):
    gsem = sems[:NBUF]
    ssem = sems[NBUF:]
    c = lax.axis_index("c")
    s = lax.axis_index("s")
    wid = c * NS + s

    zrows = N_ACC // NS
    pltpu.sync_copy(
        zeros_hbm.at[pl.ds(s * zrows, zrows)],
        acc_sh.at[pl.ds(s * zrows, zrows)],
    )

    plsc.subcore_barrier()

    @pl.loop(0, K, step=IB)
    def _(k0):
      pltpu.sync_copy(src_hbm.at[pl.ds(wid * K + k0, IB)], src_v)
      pltpu.sync_copy(dst_hbm.at[pl.ds(wid * K + k0, IB)], dst_v)
      pltpu.async_copy(xaug_hbm.at[src_v.at[0]], rows_v.at[0], gsem[0])
      scats = [None] * NBUF
      for j in range(IB):
        b = j % NBUF
        nb = (j + 1) % NBUF
        pltpu.make_async_copy(
            xaug_hbm.at[src_v.at[j]], rows_v.at[b], gsem[b]).wait()
        scats[b] = pltpu.async_copy(
            rows_v.at[b], acc_sh.at[dst_v.at[j]], ssem[b], add=True)
        if j + 1 < IB:
          if scats[nb] is not None:
            scats[nb].wait()
          pltpu.async_copy(
              xaug_hbm.at[src_v.at[j + 1]], rows_v.at[nb], gsem[nb])
      scats[(IB - 1) % NBUF].wait()

    plsc.subcore_barrier()

    orows = N_ACC // NS
    pltpu.sync_copy(
        acc_sh.at[pl.ds(s * orows, orows)],
        out_hbm.at[c, pl.ds(s * orows, orows)],
    )

  return sc_agg


BN = 2000


def _tc_body(p_ref, x_ref, w_ref, b_ref, o_ref):
  p = p_ref[0] + p_ref[1]
  deg = p[:, D:D + 1]
  agg = p[:, :D] / jnp.maximum(deg, 1.0)
  h = jnp.dot(agg, w_ref[...], preferred_element_type=jnp.float32) + b_ref[...]
  o_ref[...] = jnp.maximum(h, 0.0) + x_ref[...]


def _tc_update(partials, x, W, b2):
  return pl.pallas_call(
      _tc_body,
      grid=(N // BN,),
      in_specs=[
          pl.BlockSpec((NC, BN, DPAD), lambda i: (0, i, 0)),
          pl.BlockSpec((BN, D), lambda i: (i, 0)),
          pl.BlockSpec((D, D), lambda i: (0, 0)),
          pl.BlockSpec((1, D), lambda i: (0, 0)),
      ],
      out_specs=pl.BlockSpec((BN, D), lambda i: (i, 0)),
      out_shape=jax.ShapeDtypeStruct((N, D), jnp.float32),
  )(partials, x, W, b2)


@jax.jit
def kernel(x, edge_index, W, b):
  E = edge_index.shape[1]
  K = -(-E // (NW * CHUNK))
  K = -(-K // 8) * 8
  E_pad = NW * CHUNK * K
  pad = E_pad - E
  src = jnp.concatenate([edge_index[0], jnp.zeros((pad,), jnp.int32)])
  dst = jnp.concatenate([edge_index[1], jnp.full((pad,), N, jnp.int32)])
  src2 = src.reshape(NW * K, CHUNK)
  dst2 = dst.reshape(NW * K, CHUNK)
  xaug = jnp.concatenate(
      [x, jnp.ones((N, 1), jnp.float32), jnp.zeros((N, 15), jnp.float32)],
      axis=1)
  zeros = jnp.zeros((N_ACC, DPAD), jnp.float32)
  partials = _make_sc_agg(K)(xaug, src2, dst2, zeros)
  return _tc_update(partials, x, W, b.reshape(1, D))


# speedup vs baseline: 9.6241x; 2.2100x over previous
import functools

import jax
import jax.numpy as jnp
from jax import lax
from jax.experimental import pallas as pl
from jax.experimental.pallas import tpu as pltpu
from jax.experimental.pallas import tpu_sc as plsc

N = 10000
D = 128
DPAD = 144
N_ACC = 10112
CHUNK = 128
NC = 2
NS = 16
NW = NC * NS
NBUF = 2
IB = 8


def _make_sc_agg(K):
  mesh = plsc.VectorSubcoreMesh(core_axis_name="c", subcore_axis_name="s")

  @functools.partial(
      pl.kernel,
      mesh=mesh,
      out_type=jax.ShapeDtypeStruct((NC, N_ACC, DPAD), jnp.float32),
      scratch_types=[
          pltpu.VMEM_SHARED((N_ACC, DPAD), jnp.float32),
          pltpu.VMEM((IB, CHUNK), jnp.int32),
          pltpu.VMEM((IB, CHUNK), jnp.int32),
          pltpu.VMEM((NBUF, CHUNK, DPAD), jnp.float32),
      ] + [pltpu.SemaphoreType.DMA] * (2 * NBUF),
      compiler_params=pltpu.CompilerParams(use_tc_tiling_on_sc=False),
  )
  def sc_agg(xaug_hbm, src_hbm, dst_hbm, zeros_hbm, out_hbm, acc_sh, src_v,
             dst_v, rows_v, *sems):
    gsem = sems[:NBUF]
    ssem = sems[NBUF:]
    c = lax.axis_index("c")
    s = lax.axis_index("s")
    wid = c * NS + s

    zrows = N_ACC // NS
    pltpu.sync_copy(
        zeros_hbm.at[pl.ds(s * zrows, zrows)],
        acc_sh.at[pl.ds(s * zrows, zrows)],
    )

    plsc.subcore_barrier()

    @pl.loop(0, K, step=IB)
    def _(k0):
      pltpu.sync_copy(src_hbm.at[pl.ds(wid * K + k0, IB)], src_v)
      pltpu.sync_copy(dst_hbm.at[pl.ds(wid * K + k0, IB)], dst_v)
      pltpu.async_copy(xaug_hbm.at[src_v.at[0]], rows_v.at[0], gsem[0])
      scats = [None] * NBUF
      for j in range(IB):
        b = j % NBUF
        nb = (j + 1) % NBUF
        pltpu.make_async_copy(
            xaug_hbm.at[src_v.at[j]], rows_v.at[b], gsem[b]).wait()
        scats[b] = pltpu.async_copy(
            rows_v.at[b], acc_sh.at[dst_v.at[j]], ssem[b], add=True)
        if j + 1 < IB:
          if scats[nb] is not None:
            scats[nb].wait()
          pltpu.async_copy(
              xaug_hbm.at[src_v.at[j + 1]], rows_v.at[nb], gsem[nb])
      scats[(IB - 1) % NBUF].wait()

    plsc.subcore_barrier()

    orows = N_ACC // NS
    pltpu.sync_copy(
        acc_sh.at[pl.ds(s * orows, orows)],
        out_hbm.at[c, pl.ds(s * orows, orows)],
    )

  return sc_agg


BN = 2000


def _tc_body(p_ref, x_ref, w_ref, b_ref, o_ref):
  p = p_ref[0] + p_ref[1]
  deg = p[:, D:D + 1]
  agg = p[:, :D] / jnp.maximum(deg, 1.0)
  h = jnp.dot(agg, w_ref[...], preferred_element_type=jnp.float32) + b_ref[...]
  o_ref[...] = jnp.maximum(h, 0.0) + x_ref[...]


def _tc_update(partials, x, W, b2):
  return pl.pallas_call(
      _tc_body,
      grid=(N // BN,),
      in_specs=[
          pl.BlockSpec((NC, BN, DPAD), lambda i: (0, i, 0)),
          pl.BlockSpec((BN, D), lambda i: (i, 0)),
          pl.BlockSpec((D, D), lambda i: (0, 0)),
          pl.BlockSpec((1, D), lambda i: (0, 0)),
      ],
      out_specs=pl.BlockSpec((BN, D), lambda i: (i, 0)),
      out_shape=jax.ShapeDtypeStruct((N, D), jnp.float32),
  )(partials, x, W, b2)


@jax.jit
def kernel(x, edge_index, W, b):
  E = edge_index.shape[1]
  K = -(-E // (NW * CHUNK))
  K = -(-K // 8) * 8
  E_pad = NW * CHUNK * K
  pad = E_pad - E
  r = jnp.arange(pad, dtype=jnp.int32)
  src = jnp.concatenate([edge_index[0], r % N])
  dst = jnp.concatenate([edge_index[1], N + r % (N_ACC - N)])
  src2 = src.reshape(NW * K, CHUNK)
  dst2 = dst.reshape(NW * K, CHUNK)
  xaug = jnp.concatenate(
      [x, jnp.ones((N, 1), jnp.float32), jnp.zeros((N, 15), jnp.float32)],
      axis=1)
  zeros = jnp.zeros((N_ACC, DPAD), jnp.float32)
  partials = _make_sc_agg(K)(xaug, src2, dst2, zeros)
  return _tc_update(partials, x, W, b.reshape(1, D))
